# baseline (speedup 1.0000x reference)
import functools

import jax
import jax.numpy as jnp
from jax import lax
from jax.experimental import pallas as pl
from jax.experimental.pallas import tpu as pltpu
from jax.experimental.pallas import tpu_sc as plsc

NC = 2
NS = 16
L = 16


def _tile_body(bpw, d_latent, x_hbm, user_hbm, item_hbm, out_hbm,
               xt, idx_u, idx_i, rows_u, rows_i, out_v, sem_u, sem_i):
    wid = lax.axis_index("s") * NC + lax.axis_index("c")
    base = wid * bpw

    pltpu.sync_copy(x_hbm.at[pl.ds(base, bpw)], xt)

    lane = jnp.arange(L, dtype=jnp.int32)
    col0 = jnp.zeros((L,), jnp.int32)
    col1 = jnp.ones((L,), jnp.int32)

    def split(k, carry):
        row_idx = k * L + lane
        idx_u[pl.ds(k * L, L)] = plsc.load_gather(xt, [row_idx, col0])
        idx_i[pl.ds(k * L, L)] = plsc.load_gather(xt, [row_idx, col1])
        return carry

    lax.fori_loop(0, bpw // L, split, 0, unroll=4)

    cu = pltpu.async_copy(user_hbm.at[idx_u], rows_u, sem_u)
    ci = pltpu.async_copy(item_hbm.at[idx_i], rows_i, sem_i)
    cu.wait()
    ci.wait()

    def blk(r, carry):
        row_idx = r * L + lane
        acc = jnp.zeros((L,), jnp.float32)
        for d in range(d_latent):
            col = jnp.full((L,), d, jnp.int32)
            gu = plsc.load_gather(rows_u, [row_idx, col])
            gi = plsc.load_gather(rows_i, [row_idx, col])
            acc = acc + gu * gi
        out_v[pl.ds(r * L, L)] = acc
        return carry

    lax.fori_loop(0, bpw // L, blk, 0, unroll=2)

    pltpu.sync_copy(out_v, out_hbm.at[pl.ds(base, bpw)])


@jax.jit
def _run(x, user_table, item_table):
    b = x.shape[0]
    d_latent = user_table.shape[1]
    nw = NC * NS
    bpw = b // nw
    mesh = plsc.VectorSubcoreMesh(
        core_axis_name="c", subcore_axis_name="s",
        num_cores=NC, num_subcores=NS)
    body = functools.partial(_tile_body, bpw, d_latent)
    return pl.kernel(
        body,
        out_type=jax.ShapeDtypeStruct((b,), jnp.float32),
        mesh=mesh,
        compiler_params=pltpu.CompilerParams(needs_layout_passes=False,
                                             use_tc_tiling_on_sc=False),
        scratch_types=[
            pltpu.VMEM((bpw, 2), jnp.int32),
            pltpu.VMEM((bpw,), jnp.int32),
            pltpu.VMEM((bpw,), jnp.int32),
            pltpu.VMEM((bpw, d_latent), jnp.float32),
            pltpu.VMEM((bpw, d_latent), jnp.float32),
            pltpu.VMEM((bpw,), jnp.float32),
            pltpu.SemaphoreType.DMA,
            pltpu.SemaphoreType.DMA,
        ],
    )(x, user_table, item_table)


def kernel(x, user_table, item_table):
    return _run(x, user_table, item_table)


# speedup vs baseline: 5.3402x; 5.3402x over previous
import functools

import jax
import jax.numpy as jnp
from jax import lax
from jax.experimental import pallas as pl
from jax.experimental.pallas import tpu as pltpu
from jax.experimental.pallas import tpu_sc as plsc

NC = 2
NS = 16
L = 16
D = 16
CHR = 16


def _tile_body(bpw, ts, x_hbm, ut_hbm, it_hbm, tu_hbm, ti_hbm, out_hbm,
               xbuf, tub, tib, bufu, bufi, out_v, sem_u, sem_i):
    wid = lax.axis_index("s") * NC + lax.axis_index("c")
    base = wid * bpw
    c0max = ts - 128

    pltpu.sync_copy(x_hbm.at[pl.ds(base * 2, bpw * 2)], xbuf)
    pltpu.sync_copy(tu_hbm, tub)
    pltpu.sync_copy(ti_hbm, tib)

    lane = jnp.arange(L, dtype=jnp.int32)

    def chunk(ci, carry):
        q = ci * CHR
        uvec = plsc.load_gather(xbuf, [(q + lane) * 2])
        ivec = plsc.load_gather(xbuf, [(q + lane) * 2 + 1])
        cu0 = jnp.minimum((uvec >> 7) << 7, c0max)
        ci0 = jnp.minimum((ivec >> 7) << 7, c0max)
        cps = []
        for j in range(CHR):
            cu = pl.multiple_of(cu0[j], 128)
            cv = pl.multiple_of(ci0[j], 128)
            cps.append(pltpu.async_copy(
                ut_hbm.at[:, pl.ds(cu, 128)],
                bufu.at[pl.ds(j * D, D), :], sem_u))
            cps.append(pltpu.async_copy(
                it_hbm.at[:, pl.ds(cv, 128)],
                bufi.at[pl.ds(j * D, D), :], sem_i))
        for cp in cps:
            cp.wait()
        colu = jnp.minimum(uvec - cu0, 127)
        coli = jnp.minimum(ivec - ci0, 127)
        um = uvec >= ts
        im = ivec >= ts
        tuw = jnp.clip(uvec - ts, 0, 127)
        tiw = jnp.clip(ivec - ts, 0, 127)
        acc = jnp.zeros((L,), jnp.float32)
        for d in range(D):
            row = lane * D + d
            gu = plsc.load_gather(bufu, [row, colu])
            gi = plsc.load_gather(bufi, [row, coli])
            gtu = plsc.load_gather(tub, [d * 128 + tuw])
            gti = plsc.load_gather(tib, [d * 128 + tiw])
            u = jnp.where(um, gtu, gu)
            v = jnp.where(im, gti, gi)
            acc = acc + u * v
        out_v[pl.ds(q, L)] = acc
        return carry

    lax.fori_loop(0, bpw // CHR, chunk, 0)

    pltpu.sync_copy(out_v, out_hbm.at[pl.ds(base, bpw)])


@jax.jit
def _run(xf, ut, it, tu, ti):
    b = xf.shape[0] // 2
    nw = NC * NS
    bpw = b // nw
    ts = (ut.shape[1] >> 7) << 7
    mesh = plsc.VectorSubcoreMesh(
        core_axis_name="c", subcore_axis_name="s",
        num_cores=NC, num_subcores=NS)
    body = functools.partial(_tile_body, bpw, ts)
    return pl.kernel(
        body,
        out_type=jax.ShapeDtypeStruct((b,), jnp.float32),
        mesh=mesh,
        compiler_params=pltpu.CompilerParams(needs_layout_passes=False,
                                             use_tc_tiling_on_sc=True),
        scratch_types=[
            pltpu.VMEM((bpw * 2,), jnp.int32),
            pltpu.VMEM((D * 128,), jnp.float32),
            pltpu.VMEM((D * 128,), jnp.float32),
            pltpu.VMEM((CHR * D, 128), jnp.float32),
            pltpu.VMEM((CHR * D, 128), jnp.float32),
            pltpu.VMEM((bpw,), jnp.float32),
            pltpu.SemaphoreType.DMA,
            pltpu.SemaphoreType.DMA,
        ],
    )(xf, ut, it, tu, ti)


def _tail(table, ts):
    t = table[ts:, :]
    t = jnp.pad(t, ((0, 128 - t.shape[0]), (0, 0)))
    return t.T.reshape(-1)


def kernel(x, user_table, item_table):
    ts = (user_table.shape[0] >> 7) << 7
    tu = _tail(user_table, ts)
    ti = _tail(item_table, ts)
    return _run(x.reshape(-1), user_table.T, item_table.T, tu, ti)
